# baseline (speedup 1.0000x reference)
import functools

import jax
import jax.numpy as jnp
from jax import lax
from jax.experimental import pallas as pl
from jax.experimental.pallas import tpu as pltpu
from jax.experimental.pallas import tpu_sc as plsc

DEPTH = 64
B = 16384
MARGIN = 1.0

E_ROWS = 4 * B
R_ROWS = 2 * B
CH = 128

NC, NS = 2, 16
NW = NC * NS

E_PER_W = E_ROWS // NW
R_PER_W = R_ROWS // NW
E_CHUNKS = E_PER_W // CH
R_CHUNKS = R_PER_W // CH


def _sc_gather(ent_hbm, rel_hbm, eidx_hbm, ridx_hbm, eout_hbm, rout_hbm,
               eidx_v, ridx_v, buf_v, sem):
    wid = lax.axis_index("s") * NC + lax.axis_index("c")
    ebase = wid * E_PER_W
    rbase = wid * R_PER_W

    pltpu.sync_copy(eidx_hbm.at[pl.ds(wid * E_CHUNKS, E_CHUNKS)], eidx_v)
    pltpu.sync_copy(ridx_hbm.at[pl.ds(wid * R_CHUNKS, R_CHUNKS)], ridx_v)

    def ebody(j, carry):
        pltpu.async_copy(ent_hbm.at[eidx_v.at[j]], buf_v, sem).wait()
        pltpu.sync_copy(buf_v, eout_hbm.at[pl.ds(ebase + j * CH, CH)])
        return carry

    lax.fori_loop(0, E_CHUNKS, ebody, 0, unroll=False)

    def rbody(j, carry):
        pltpu.async_copy(rel_hbm.at[ridx_v.at[j]], buf_v, sem).wait()
        pltpu.sync_copy(buf_v, rout_hbm.at[pl.ds(rbase + j * CH, CH)])
        return carry

    lax.fori_loop(0, R_CHUNKS, rbody, 0, unroll=False)


@functools.cache
def _gather_call():
    return pl.kernel(
        _sc_gather,
        out_type=[
            jax.ShapeDtypeStruct((E_ROWS, DEPTH), jnp.float32),
            jax.ShapeDtypeStruct((R_ROWS, DEPTH), jnp.float32),
        ],
        mesh=plsc.VectorSubcoreMesh(core_axis_name="c", subcore_axis_name="s"),
        compiler_params=pltpu.CompilerParams(use_tc_tiling_on_sc=False),
        scratch_types=[
            pltpu.VMEM((E_CHUNKS, CH), jnp.int32),
            pltpu.VMEM((R_CHUNKS, CH), jnp.int32),
            pltpu.VMEM((CH, DEPTH), jnp.float32),
            pltpu.SemaphoreType.DMA,
        ],
    )


BLK = 2048
GRID = B // BLK


def _tc_body(ph, pt, nh, nt, pr, nr, l1, l2, bl, r1, r2, br, out_ref):
    i = pl.program_id(0)

    def norm(x):
        ss = jnp.sum(x * x, axis=1, keepdims=True)
        return x / (jnp.sqrt(ss) + 1e-12)

    def score(h, t, r):
        lo = (jnp.dot(norm(h), l1[...], preferred_element_type=jnp.float32)
              + jnp.dot(r, l2[...], preferred_element_type=jnp.float32)
              + bl[...])
        ro = (jnp.dot(norm(t), r1[...], preferred_element_type=jnp.float32)
              + jnp.dot(r, r2[...], preferred_element_type=jnp.float32)
              + br[...])
        return jnp.sum(lo * ro, axis=1)

    s_pos = score(ph[...], pt[...], pr[...])
    s_neg = score(nh[...], nt[...], nr[...])
    part = jnp.sum(jnp.maximum(MARGIN + s_neg - s_pos, 0.0))

    @pl.when(i == 0)
    def _():
        out_ref[0, 0] = 0.0

    out_ref[0, 0] += part

    @pl.when(i == GRID - 1)
    def _():
        out_ref[0, 0] = out_ref[0, 0] * (1.0 / B)


def _row_spec(block_off):
    return pl.BlockSpec((BLK, DEPTH), lambda i, o=block_off: (i + o, 0))


def _full_spec(shape):
    return pl.BlockSpec(shape, lambda i: (0, 0))


def kernel(pos_x, neg_x, ent_emb, rel_emb, lll_lmat, lll_rmat, lll_bias,
           rll_lmat, rll_rmat, rll_bias):
    eidx = jnp.concatenate(
        [pos_x[:, 0], pos_x[:, 1], neg_x[:, 0], neg_x[:, 1]]
    ).astype(jnp.int32).reshape(E_ROWS // CH, CH)
    ridx = jnp.concatenate([pos_x[:, 2], neg_x[:, 2]]).astype(
        jnp.int32).reshape(R_ROWS // CH, CH)

    ent_rows, rel_rows = _gather_call()(ent_emb, rel_emb, eidx, ridx)

    nblk = GRID
    out = pl.pallas_call(
        _tc_body,
        grid=(GRID,),
        in_specs=[
            _row_spec(0),
            _row_spec(nblk),
            _row_spec(2 * nblk),
            _row_spec(3 * nblk),
            _row_spec(0),
            _row_spec(nblk),
            _full_spec((DEPTH, DEPTH)),
            _full_spec((DEPTH, DEPTH)),
            _full_spec((1, DEPTH)),
            _full_spec((DEPTH, DEPTH)),
            _full_spec((DEPTH, DEPTH)),
            _full_spec((1, DEPTH)),
        ],
        out_specs=pl.BlockSpec((1, 1), lambda i: (0, 0),
                               memory_space=pltpu.SMEM),
        out_shape=jax.ShapeDtypeStruct((1, 1), jnp.float32),
    )(ent_rows, ent_rows, ent_rows, ent_rows, rel_rows, rel_rows,
      lll_lmat, lll_rmat, lll_bias, rll_lmat, rll_rmat, rll_bias)

    return out[0, 0]


# speedup vs baseline: 1.0122x; 1.0122x over previous
import functools

import jax
import jax.numpy as jnp
from jax import lax
from jax.experimental import pallas as pl
from jax.experimental.pallas import tpu as pltpu
from jax.experimental.pallas import tpu_sc as plsc

DEPTH = 64
B = 16384
MARGIN = 1.0

E_ROWS = 4 * B
R_ROWS = 2 * B
CH = 512

NC, NS = 2, 16
NW = NC * NS

E_PER_W = E_ROWS // NW
R_PER_W = R_ROWS // NW
E_CHUNKS = E_PER_W // CH
R_CHUNKS = R_PER_W // CH


def _sc_gather(ent_hbm, rel_hbm, eidx_hbm, ridx_hbm, eout_hbm, rout_hbm,
               eidx_v, ridx_v, buf_v, sem):
    wid = lax.axis_index("s") * NC + lax.axis_index("c")
    ebase = wid * E_PER_W
    rbase = wid * R_PER_W

    pltpu.sync_copy(eidx_hbm.at[pl.ds(wid * E_CHUNKS, E_CHUNKS)], eidx_v)
    pltpu.sync_copy(ridx_hbm.at[pl.ds(wid * R_CHUNKS, R_CHUNKS)], ridx_v)

    def ebody(j, carry):
        pltpu.async_copy(ent_hbm.at[eidx_v.at[j]], buf_v, sem).wait()
        pltpu.sync_copy(buf_v, eout_hbm.at[pl.ds(ebase + j * CH, CH)])
        return carry

    lax.fori_loop(0, E_CHUNKS, ebody, 0, unroll=False)

    def rbody(j, carry):
        pltpu.async_copy(rel_hbm.at[ridx_v.at[j]], buf_v, sem).wait()
        pltpu.sync_copy(buf_v, rout_hbm.at[pl.ds(rbase + j * CH, CH)])
        return carry

    lax.fori_loop(0, R_CHUNKS, rbody, 0, unroll=False)


@functools.cache
def _gather_call():
    return pl.kernel(
        _sc_gather,
        out_type=[
            jax.ShapeDtypeStruct((E_ROWS, DEPTH), jnp.float32),
            jax.ShapeDtypeStruct((R_ROWS, DEPTH), jnp.float32),
        ],
        mesh=plsc.VectorSubcoreMesh(core_axis_name="c", subcore_axis_name="s"),
        compiler_params=pltpu.CompilerParams(use_tc_tiling_on_sc=False),
        scratch_types=[
            pltpu.VMEM((E_CHUNKS, CH), jnp.int32),
            pltpu.VMEM((R_CHUNKS, CH), jnp.int32),
            pltpu.VMEM((CH, DEPTH), jnp.float32),
            pltpu.SemaphoreType.DMA,
        ],
    )


BLK = 2048
GRID = B // BLK


def _tc_body(ph, pt, nh, nt, pr, nr, l1, l2, bl, r1, r2, br, out_ref):
    i = pl.program_id(0)

    def norm(x):
        ss = jnp.sum(x * x, axis=1, keepdims=True)
        return x / (jnp.sqrt(ss) + 1e-12)

    def score(h, t, r):
        lo = (jnp.dot(norm(h), l1[...], preferred_element_type=jnp.float32)
              + jnp.dot(r, l2[...], preferred_element_type=jnp.float32)
              + bl[...])
        ro = (jnp.dot(norm(t), r1[...], preferred_element_type=jnp.float32)
              + jnp.dot(r, r2[...], preferred_element_type=jnp.float32)
              + br[...])
        return jnp.sum(lo * ro, axis=1)

    s_pos = score(ph[...], pt[...], pr[...])
    s_neg = score(nh[...], nt[...], nr[...])
    part = jnp.sum(jnp.maximum(MARGIN + s_neg - s_pos, 0.0))

    @pl.when(i == 0)
    def _():
        out_ref[0, 0] = 0.0

    out_ref[0, 0] += part

    @pl.when(i == GRID - 1)
    def _():
        out_ref[0, 0] = out_ref[0, 0] * (1.0 / B)


def _row_spec(block_off):
    return pl.BlockSpec((BLK, DEPTH), lambda i, o=block_off: (i + o, 0))


def _full_spec(shape):
    return pl.BlockSpec(shape, lambda i: (0, 0))


def kernel(pos_x, neg_x, ent_emb, rel_emb, lll_lmat, lll_rmat, lll_bias,
           rll_lmat, rll_rmat, rll_bias):
    eidx = jnp.concatenate(
        [pos_x[:, 0], pos_x[:, 1], neg_x[:, 0], neg_x[:, 1]]
    ).astype(jnp.int32).reshape(E_ROWS // CH, CH)
    ridx = jnp.concatenate([pos_x[:, 2], neg_x[:, 2]]).astype(
        jnp.int32).reshape(R_ROWS // CH, CH)

    ent_rows, rel_rows = _gather_call()(ent_emb, rel_emb, eidx, ridx)

    nblk = GRID
    out = pl.pallas_call(
        _tc_body,
        grid=(GRID,),
        in_specs=[
            _row_spec(0),
            _row_spec(nblk),
            _row_spec(2 * nblk),
            _row_spec(3 * nblk),
            _row_spec(0),
            _row_spec(nblk),
            _full_spec((DEPTH, DEPTH)),
            _full_spec((DEPTH, DEPTH)),
            _full_spec((1, DEPTH)),
            _full_spec((DEPTH, DEPTH)),
            _full_spec((DEPTH, DEPTH)),
            _full_spec((1, DEPTH)),
        ],
        out_specs=pl.BlockSpec((1, 1), lambda i: (0, 0),
                               memory_space=pltpu.SMEM),
        out_shape=jax.ShapeDtypeStruct((1, 1), jnp.float32),
    )(ent_rows, ent_rows, ent_rows, ent_rows, rel_rows, rel_rows,
      lll_lmat, lll_rmat, lll_bias, rll_lmat, rll_rmat, rll_bias)

    return out[0, 0]
